# baseline (speedup 1.0000x reference)
import functools

import jax
import jax.numpy as jnp
from jax import lax
from jax.experimental import pallas as pl
from jax.experimental.pallas import tpu as pltpu
from jax.experimental.pallas import tpu_sc as plsc

B, S, D = 4, 8192, 2048
S_BLK = 512
NS = S // S_BLK

NC, NSUB, L = 2, 16, 16
NW = NC * NSUB
ROWS_PER_W = (B * S) // NW
CHUNKS = ROWS_PER_W // 128


def _copy_reduce_kernel(idx_ref, x_ref, out_ref, vals_ref, acc_ref, mins_ref):
    j = pl.program_id(0)
    s = pl.program_id(1)
    blk = x_ref[0]
    out_ref[0] = blk
    part = jnp.min(blk, axis=0, keepdims=True)

    @pl.when(s == 0)
    def _():
        acc_ref[...] = part

    @pl.when(s != 0)
    def _():
        acc_ref[...] = jnp.minimum(acc_ref[...], part)

    @pl.when(s == NS - 1)
    def _():
        acc = acc_ref[...]
        idx = idx_ref[j]
        lane = jax.lax.broadcasted_iota(jnp.int32, (1, D), 1)
        mins_ref[j, 0] = jnp.min(acc)
        mins_ref[j, 1] = jnp.min(jnp.where(lane == idx, jnp.inf, acc))

    @pl.when((j == B - 1) & (s == NS - 1))
    def _():
        prefix = jnp.float32(jnp.inf)
        for i in range(B):
            suf = mins_ref[i, 0]
            for k in range(i + 1, B):
                suf = jnp.minimum(suf, mins_ref[k, 0])
            m = jnp.minimum(prefix, suf)
            v = jnp.where(m == 0.0, jnp.float32(0.0), m - jnp.float32(100000.0))
            vals_ref[i] = v
            prefix = jnp.minimum(prefix, jnp.minimum(mins_ref[i, 1], v))


def _sc_fixup_body(idx_hbm, vals_hbm, big_ref, idx_v, vals_v, val128, *rest):
    addr_refs, sem = rest[:CHUNKS], rest[CHUNKS]
    w = lax.axis_index("s") * NC + lax.axis_index("c")
    i = w // (NW // B)
    r0 = (w % (NW // B)) * ROWS_PER_W
    pltpu.sync_copy(idx_hbm, idx_v)
    pltpu.sync_copy(vals_hbm, vals_v)
    sel = jnp.zeros((L,), jnp.int32) + i
    idx_b = jnp.take_along_axis(idx_v[...], sel, axis=0)
    val_b = jnp.take_along_axis(vals_v[...], sel, axis=0)
    iota = lax.iota(jnp.int32, L)
    base = i * (S * D) + r0 * D + iota * D + idx_b
    for t in range(128 // L):
        val128[pl.ds(t * L, L)] = val_b
    for j in range(CHUNKS):
        for t in range(128 // L):
            g = j * (128 // L) + t
            addr_refs[j][pl.ds(t * L, L)] = base + (g * L) * D
    for j in range(CHUNKS):
        pltpu.make_async_copy(val128, big_ref.at[addr_refs[j]], sem).start()
    for j in range(CHUNKS):
        pltpu.make_async_copy(val128, big_ref.at[addr_refs[j]], sem).wait()


def kernel(x, indices):
    indices = indices.astype(jnp.int32)
    big, vals = pl.pallas_call(
        _copy_reduce_kernel,
        grid_spec=pltpu.PrefetchScalarGridSpec(
            num_scalar_prefetch=1,
            grid=(B, NS),
            in_specs=[
                pl.BlockSpec((1, S_BLK, D), lambda j, s, idx: (j, s, 0)),
            ],
            out_specs=[
                pl.BlockSpec((1, S_BLK, D), lambda j, s, idx: (j, s, 0)),
                pl.BlockSpec(memory_space=pltpu.SMEM),
            ],
            scratch_shapes=[
                pltpu.VMEM((1, D), jnp.float32),
                pltpu.SMEM((B, 2), jnp.float32),
            ],
        ),
        out_shape=[
            jax.ShapeDtypeStruct((B, S, D), jnp.float32),
            jax.ShapeDtypeStruct((B,), jnp.float32),
        ],
    )(indices, x)

    idx_pad = jnp.zeros((L,), jnp.int32).at[:B].set(indices)
    vals_pad = jnp.zeros((L,), jnp.float32).at[:B].set(vals)
    mesh = plsc.VectorSubcoreMesh(core_axis_name="c", subcore_axis_name="s")
    ref = jax.new_ref(big.reshape(-1))
    sc_fixup = pl.kernel(
        _sc_fixup_body,
        out_type=(),
        mesh=mesh,
        scratch_types=[
            pltpu.VMEM((L,), jnp.int32),
            pltpu.VMEM((L,), jnp.float32),
            pltpu.VMEM((128,), jnp.float32),
            *[pltpu.VMEM((128,), jnp.int32) for _ in range(CHUNKS)],
            pltpu.SemaphoreType.DMA,
        ],
    )
    sc_fixup(idx_pad, vals_pad, ref)
    return ref[...].reshape(B, S, D)


# speedup vs baseline: 3.7369x; 3.7369x over previous
import jax
import jax.numpy as jnp
from jax.experimental import pallas as pl
from jax.experimental.pallas import tpu as pltpu

B, S, D = 4, 8192, 2048
S_BLK = 1024
NS = S // S_BLK
LANES = 128


def _copy_reduce_kernel(idx_ref, x_ref, out_ref, vals_ref, acc_ref, mins_ref):
    j = pl.program_id(0)
    s = pl.program_id(1)
    blk = x_ref[0]
    out_ref[0] = blk
    part = jnp.min(blk, axis=0, keepdims=True)

    @pl.when(s == 0)
    def _():
        acc_ref[...] = part

    @pl.when(s != 0)
    def _():
        acc_ref[...] = jnp.minimum(acc_ref[...], part)

    @pl.when(s == NS - 1)
    def _():
        acc = acc_ref[...]
        idx = idx_ref[j]
        lane = jax.lax.broadcasted_iota(jnp.int32, (1, D), 1)
        mins_ref[j, 0] = jnp.min(acc)
        mins_ref[j, 1] = jnp.min(jnp.where(lane == idx, jnp.inf, acc))

    @pl.when((j == B - 1) & (s == NS - 1))
    def _():
        prefix = jnp.float32(jnp.inf)
        for i in range(B):
            suf = mins_ref[i, 0]
            for k in range(i + 1, B):
                suf = jnp.minimum(suf, mins_ref[k, 0])
            m = jnp.minimum(prefix, suf)
            v = jnp.where(m == 0.0, jnp.float32(0.0), m - jnp.float32(100000.0))
            vals_ref[i] = v
            prefix = jnp.minimum(prefix, jnp.minimum(mins_ref[i, 1], v))


def _fixup_kernel(idx_ref, vals_ref, big_ref, out_ref):
    i = pl.program_id(0)
    v = vals_ref[i]
    col = idx_ref[i] % LANES
    lane = jax.lax.broadcasted_iota(jnp.int32, (1, S, LANES), 2)
    out_ref[...] = jnp.where(lane == col, v, big_ref[...])


def kernel(x, indices):
    indices = indices.astype(jnp.int32)
    big, vals = pl.pallas_call(
        _copy_reduce_kernel,
        grid_spec=pltpu.PrefetchScalarGridSpec(
            num_scalar_prefetch=1,
            grid=(B, NS),
            in_specs=[
                pl.BlockSpec((1, S_BLK, D), lambda j, s, idx: (j, s, 0)),
            ],
            out_specs=[
                pl.BlockSpec((1, S_BLK, D), lambda j, s, idx: (j, s, 0)),
                pl.BlockSpec(memory_space=pltpu.SMEM),
            ],
            scratch_shapes=[
                pltpu.VMEM((1, D), jnp.float32),
                pltpu.SMEM((B, 2), jnp.float32),
            ],
        ),
        out_shape=[
            jax.ShapeDtypeStruct((B, S, D), jnp.float32),
            jax.ShapeDtypeStruct((B,), jnp.float32),
        ],
    )(indices, x)

    out = pl.pallas_call(
        _fixup_kernel,
        grid_spec=pltpu.PrefetchScalarGridSpec(
            num_scalar_prefetch=1,
            grid=(B,),
            in_specs=[
                pl.BlockSpec(memory_space=pltpu.SMEM),
                pl.BlockSpec((1, S, LANES), lambda i, idx: (i, 0, idx[i] // LANES)),
            ],
            out_specs=pl.BlockSpec(
                (1, S, LANES), lambda i, idx: (i, 0, idx[i] // LANES)
            ),
        ),
        out_shape=jax.ShapeDtypeStruct((B, S, D), jnp.float32),
        input_output_aliases={2: 0},
    )(indices, vals, big)
    return out
